# baseline (speedup 1.0000x reference)
import jax
import jax.numpy as jnp
from jax import lax
from jax.experimental import pallas as pl
from jax.experimental.pallas import tpu as pltpu
from jax.experimental.pallas import tpu_sc as plsc

BSZ = 16384
FEA = 64
NC = 2
NS = 16
NW = NC * NS
B_PER_W = BSZ // NW
LANES = 16
PITCH = LANES + 1
NGROUP = B_PER_W // LANES


def _sc_kernel(ufea_hbm, ifea_hbm, uid_hbm, iid_hbm, go_hbm, utab_hbm,
               itab_hbm, out_hbm,
               uidx_v, iidx_v, uoff_v, ioff_v, ufea_v, ifea_v, psum_v,
               res_v, go_v, sem_u, sem_i, sem_uf, sem_if, sem_g):
    wid = lax.axis_index("s") * NC + lax.axis_index("c")
    base = wid * B_PER_W
    pltpu.sync_copy(uid_hbm.at[pl.ds(base, B_PER_W)], uidx_v)
    pltpu.sync_copy(iid_hbm.at[pl.ds(base, B_PER_W)], iidx_v)
    cp_g = pltpu.async_copy(go_hbm.at[jnp.zeros((LANES,), jnp.int32)],
                            go_v, sem_g)
    cp_u = pltpu.async_copy(utab_hbm.at[uidx_v], uoff_v, sem_u)
    cp_i = pltpu.async_copy(itab_hbm.at[iidx_v], ioff_v, sem_i)
    cp_uf = pltpu.async_copy(ufea_hbm.at[pl.ds(base, B_PER_W)], ufea_v, sem_uf)
    cp_if = pltpu.async_copy(ifea_hbm.at[pl.ds(base, B_PER_W)], ifea_v, sem_if)
    cp_uf.wait()
    cp_if.wait()

    def _row(r, carry):
        p = (ufea_v[r, pl.ds(0, LANES)] * ifea_v[r, pl.ds(0, LANES)]
             + ufea_v[r, pl.ds(LANES, LANES)] * ifea_v[r, pl.ds(LANES, LANES)]
             + ufea_v[r, pl.ds(2 * LANES, LANES)] * ifea_v[r, pl.ds(2 * LANES, LANES)]
             + ufea_v[r, pl.ds(3 * LANES, LANES)] * ifea_v[r, pl.ds(3 * LANES, LANES)])
        psum_v[pl.ds(r * PITCH, LANES)] = p
        return carry

    lax.fori_loop(0, B_PER_W, _row, 0, unroll=4)

    cp_u.wait()
    cp_i.wait()
    cp_g.wait()
    iota = lax.iota(jnp.int32, LANES)
    go16 = go_v[...]

    for g in range(NGROUP):
        gbase = g * (LANES * PITCH)
        acc = go16
        for j in range(LANES):
            acc = acc + plsc.load_gather(psum_v, [iota * PITCH + (gbase + j)])
        row = g * LANES
        res_v[pl.ds(row, LANES)] = (acc + uoff_v[pl.ds(row, LANES)]
                                    + ioff_v[pl.ds(row, LANES)])

    pltpu.sync_copy(res_v, out_hbm.at[pl.ds(base, B_PER_W)])


def kernel(batch_userFea, batch_itemFea, batch_uid, batch_iid,
           globalOffset, uid_userOffset, iid_itemOffset):
    mesh = plsc.VectorSubcoreMesh(
        core_axis_name="c", subcore_axis_name="s",
        num_cores=NC, num_subcores=NS)
    out = pl.kernel(
        _sc_kernel,
        out_type=jax.ShapeDtypeStruct((BSZ,), jnp.float32),
        mesh=mesh,
        compiler_params=pltpu.CompilerParams(
            needs_layout_passes=False, use_tc_tiling_on_sc=False),
        scratch_types=[
            pltpu.VMEM((B_PER_W,), jnp.int32),
            pltpu.VMEM((B_PER_W,), jnp.int32),
            pltpu.VMEM((B_PER_W,), jnp.float32),
            pltpu.VMEM((B_PER_W,), jnp.float32),
            pltpu.VMEM((B_PER_W, FEA), jnp.float32),
            pltpu.VMEM((B_PER_W, FEA), jnp.float32),
            pltpu.VMEM((B_PER_W * PITCH,), jnp.float32),
            pltpu.VMEM((B_PER_W,), jnp.float32),
            pltpu.VMEM((LANES,), jnp.float32),
            pltpu.SemaphoreType.DMA,
            pltpu.SemaphoreType.DMA,
            pltpu.SemaphoreType.DMA,
            pltpu.SemaphoreType.DMA,
            pltpu.SemaphoreType.DMA,
        ],
    )(batch_userFea, batch_itemFea,
      batch_uid.astype(jnp.int32), batch_iid.astype(jnp.int32),
      globalOffset, uid_userOffset.reshape(-1), iid_itemOffset.reshape(-1))
    return out.reshape(BSZ, 1)


# speedup vs baseline: 1.3034x; 1.3034x over previous
import jax
import jax.numpy as jnp
from jax import lax
from jax.experimental import pallas as pl
from jax.experimental.pallas import tpu as pltpu
from jax.experimental.pallas import tpu_sc as plsc

BSZ = 16384
FEA = 64
NC = 2
NS = 16
NW = NC * NS
B_PER_W = BSZ // NW
LANES = 16


def _sc_gather_kernel(uid_hbm, iid_hbm, utab_hbm, itab_hbm,
                      uout_hbm, iout_hbm,
                      uidx_v, iidx_v, urows_v, irows_v, sem_u, sem_i):
    wid = lax.axis_index("s") * NC + lax.axis_index("c")
    base = wid * B_PER_W
    pltpu.sync_copy(uid_hbm.at[pl.ds(base, B_PER_W)], uidx_v)
    pltpu.sync_copy(iid_hbm.at[pl.ds(base, B_PER_W)], iidx_v)
    cp_u = pltpu.async_copy(utab_hbm.at[0].at[uidx_v], urows_v, sem_u)
    cp_i = pltpu.async_copy(itab_hbm.at[0].at[iidx_v], irows_v, sem_i)
    cp_u.wait()
    cp_i.wait()
    pltpu.sync_copy(urows_v, uout_hbm.at[pl.ds(base, B_PER_W)])
    pltpu.sync_copy(irows_v, iout_hbm.at[pl.ds(base, B_PER_W)])


def _sc_gather(uid, iid, utab, itab):
    mesh = plsc.VectorSubcoreMesh(
        core_axis_name="c", subcore_axis_name="s",
        num_cores=NC, num_subcores=NS)
    return pl.kernel(
        _sc_gather_kernel,
        out_type=(jax.ShapeDtypeStruct((BSZ,), jnp.float32),
                  jax.ShapeDtypeStruct((BSZ,), jnp.float32)),
        mesh=mesh,
        compiler_params=pltpu.CompilerParams(
            needs_layout_passes=False, use_tc_tiling_on_sc=False),
        scratch_types=[
            pltpu.VMEM((B_PER_W,), jnp.int32),
            pltpu.VMEM((B_PER_W,), jnp.int32),
            pltpu.VMEM((B_PER_W,), jnp.float32),
            pltpu.VMEM((B_PER_W,), jnp.float32),
            pltpu.SemaphoreType.DMA,
            pltpu.SemaphoreType.DMA,
        ],
    )(uid, iid, utab, itab)


NUM_ROWS = 1000000
PAD_ROWS = 1000448


def _tc_repack_kernel(u_ref, i_ref, uo_ref, io_ref, su, si):
    cu = pltpu.make_async_copy(u_ref, uo_ref.at[pl.ds(0, NUM_ROWS), :], su)
    ci = pltpu.make_async_copy(i_ref, io_ref.at[pl.ds(0, NUM_ROWS), :], si)
    cu.start()
    ci.start()
    cu.wait()
    ci.wait()


def _tc_repack(utab, itab):
    return pl.pallas_call(
        _tc_repack_kernel,
        in_specs=[
            pl.BlockSpec(memory_space=pl.ANY),
            pl.BlockSpec(memory_space=pl.ANY),
        ],
        out_specs=[
            pl.BlockSpec(memory_space=pl.ANY),
            pl.BlockSpec(memory_space=pl.ANY),
        ],
        out_shape=[
            jax.ShapeDtypeStruct((PAD_ROWS, 1), jnp.float32),
            jax.ShapeDtypeStruct((PAD_ROWS, 1), jnp.float32),
        ],
        scratch_shapes=[pltpu.SemaphoreType.DMA, pltpu.SemaphoreType.DMA],
    )(utab, itab)


def _tc_combine_kernel(go_ref, u_ref, i_ref, ub_ref, ib_ref, o_ref):
    dot = jnp.sum(u_ref[...] * i_ref[...], axis=0)
    o_ref[...] = dot + ub_ref[...] + ib_ref[...] + go_ref[0]


def _tc_combine(ufea_t, ifea_t, ubias, ibias, globalOffset):
    return pl.pallas_call(
        _tc_combine_kernel,
        out_shape=jax.ShapeDtypeStruct((BSZ,), jnp.float32),
    )(globalOffset, ufea_t, ifea_t, ubias, ibias)


def kernel(batch_userFea, batch_itemFea, batch_uid, batch_iid,
           globalOffset, uid_userOffset, iid_itemOffset):
    utab, itab = lax.optimization_barrier(
        (uid_userOffset.T, iid_itemOffset.T))
    ubias, ibias = _sc_gather(batch_uid.astype(jnp.int32),
                              batch_iid.astype(jnp.int32), utab, itab)
    out = _tc_combine(batch_userFea.T, batch_itemFea.T,
                      ubias, ibias, globalOffset)
    return out.reshape(BSZ, 1)


# speedup vs baseline: 5.3943x; 4.1386x over previous
import jax
import jax.numpy as jnp
from jax import lax
from jax.experimental import pallas as pl
from jax.experimental.pallas import tpu as pltpu
from jax.experimental.pallas import tpu_sc as plsc

BSZ = 16384
FEA = 64
NC = 2
NS = 16
NW = NC * NS
B_PER_W = BSZ // NW
LANES = 16


def _sc_gather_kernel(uid_hbm, iid_hbm, utab_hbm, itab_hbm,
                      uout_hbm, iout_hbm,
                      uidx_v, iidx_v, urows_v, irows_v, sem_u, sem_i):
    wid = lax.axis_index("s") * NC + lax.axis_index("c")
    base = wid * B_PER_W
    pltpu.sync_copy(uid_hbm.at[pl.ds(base, B_PER_W)], uidx_v)
    pltpu.sync_copy(iid_hbm.at[pl.ds(base, B_PER_W)], iidx_v)
    cp_u = pltpu.async_copy(utab_hbm.at[0].at[uidx_v], urows_v, sem_u)
    cp_i = pltpu.async_copy(itab_hbm.at[0].at[iidx_v], irows_v, sem_i)
    cp_u.wait()
    cp_i.wait()
    pltpu.sync_copy(urows_v, uout_hbm.at[pl.ds(base, B_PER_W)])
    pltpu.sync_copy(irows_v, iout_hbm.at[pl.ds(base, B_PER_W)])


def _sc_gather(uid, iid, utab, itab):
    mesh = plsc.VectorSubcoreMesh(
        core_axis_name="c", subcore_axis_name="s",
        num_cores=NC, num_subcores=NS)
    return pl.kernel(
        _sc_gather_kernel,
        out_type=(jax.ShapeDtypeStruct((BSZ,), jnp.float32),
                  jax.ShapeDtypeStruct((BSZ,), jnp.float32)),
        mesh=mesh,
        compiler_params=pltpu.CompilerParams(
            needs_layout_passes=False, use_tc_tiling_on_sc=True),
        scratch_types=[
            pltpu.VMEM((B_PER_W,), jnp.int32),
            pltpu.VMEM((B_PER_W,), jnp.int32),
            pltpu.VMEM((B_PER_W,), jnp.float32),
            pltpu.VMEM((B_PER_W,), jnp.float32),
            pltpu.SemaphoreType.DMA,
            pltpu.SemaphoreType.DMA,
        ],
    )(uid, iid, utab, itab)


NUM_ROWS = 1000000
PAD_ROWS = 1000448


def _tc_repack_kernel(u_ref, i_ref, uo_ref, io_ref, su, si):
    cu = pltpu.make_async_copy(u_ref, uo_ref.at[pl.ds(0, NUM_ROWS), :], su)
    ci = pltpu.make_async_copy(i_ref, io_ref.at[pl.ds(0, NUM_ROWS), :], si)
    cu.start()
    ci.start()
    cu.wait()
    ci.wait()


def _tc_repack(utab, itab):
    return pl.pallas_call(
        _tc_repack_kernel,
        in_specs=[
            pl.BlockSpec(memory_space=pl.ANY),
            pl.BlockSpec(memory_space=pl.ANY),
        ],
        out_specs=[
            pl.BlockSpec(memory_space=pl.ANY),
            pl.BlockSpec(memory_space=pl.ANY),
        ],
        out_shape=[
            jax.ShapeDtypeStruct((PAD_ROWS, 1), jnp.float32),
            jax.ShapeDtypeStruct((PAD_ROWS, 1), jnp.float32),
        ],
        scratch_shapes=[pltpu.SemaphoreType.DMA, pltpu.SemaphoreType.DMA],
    )(utab, itab)


def _tc_combine_kernel(go_ref, u_ref, i_ref, ub_ref, ib_ref, o_ref):
    dot = jnp.sum(u_ref[...] * i_ref[...], axis=0)
    o_ref[...] = dot + ub_ref[...] + ib_ref[...] + go_ref[0]


def _tc_combine(ufea_t, ifea_t, ubias, ibias, globalOffset):
    return pl.pallas_call(
        _tc_combine_kernel,
        out_shape=jax.ShapeDtypeStruct((BSZ,), jnp.float32),
    )(globalOffset, ufea_t, ifea_t, ubias, ibias)


def kernel(batch_userFea, batch_itemFea, batch_uid, batch_iid,
           globalOffset, uid_userOffset, iid_itemOffset):
    utab, itab = lax.optimization_barrier(
        (uid_userOffset.T, iid_itemOffset.T))
    ubias, ibias = _sc_gather(batch_uid.astype(jnp.int32),
                              batch_iid.astype(jnp.int32), utab, itab)
    out = _tc_combine(batch_userFea.T, batch_itemFea.T,
                      ubias, ibias, globalOffset)
    return out.reshape(BSZ, 1)


# speedup vs baseline: 5.8862x; 1.0912x over previous
import jax
import jax.numpy as jnp
from jax import lax
from jax.experimental import pallas as pl
from jax.experimental.pallas import tpu as pltpu
from jax.experimental.pallas import tpu_sc as plsc

BSZ = 16384
FEA = 64
NC = 2
NS = 16
NW = NC * NS
B_PER_W = BSZ // NW
LANES = 16


def _sc_gather_kernel(uid_hbm, iid_hbm, utab_hbm, itab_hbm,
                      uout_hbm, iout_hbm,
                      uidx_v, iidx_v, urows_v, irows_v, sem_u, sem_i):
    wid = lax.axis_index("s") * NC + lax.axis_index("c")
    base = wid * B_PER_W
    pltpu.sync_copy(uid_hbm.at[pl.ds(base, B_PER_W)], uidx_v)
    pltpu.sync_copy(iid_hbm.at[pl.ds(base, B_PER_W)], iidx_v)
    cp_u = pltpu.async_copy(utab_hbm.at[0].at[uidx_v], urows_v, sem_u)
    cp_i = pltpu.async_copy(itab_hbm.at[0].at[iidx_v], irows_v, sem_i)
    cp_u.wait()
    cp_i.wait()
    pltpu.sync_copy(urows_v, uout_hbm.at[pl.ds(base, B_PER_W)])
    pltpu.sync_copy(irows_v, iout_hbm.at[pl.ds(base, B_PER_W)])


def _sc_gather(uid, iid, utab, itab):
    mesh = plsc.VectorSubcoreMesh(
        core_axis_name="c", subcore_axis_name="s",
        num_cores=NC, num_subcores=NS)
    return pl.kernel(
        _sc_gather_kernel,
        out_type=(jax.ShapeDtypeStruct((BSZ,), jnp.float32),
                  jax.ShapeDtypeStruct((BSZ,), jnp.float32)),
        mesh=mesh,
        compiler_params=pltpu.CompilerParams(
            needs_layout_passes=False, use_tc_tiling_on_sc=True),
        scratch_types=[
            pltpu.VMEM((B_PER_W,), jnp.int32),
            pltpu.VMEM((B_PER_W,), jnp.int32),
            pltpu.VMEM((B_PER_W,), jnp.float32),
            pltpu.VMEM((B_PER_W,), jnp.float32),
            pltpu.SemaphoreType.DMA,
            pltpu.SemaphoreType.DMA,
        ],
    )(uid, iid, utab, itab)


NUM_ROWS = 1000000
PAD_ROWS = 1000448


def _tc_repack_kernel(u_ref, i_ref, uo_ref, io_ref, su, si):
    cu = pltpu.make_async_copy(u_ref, uo_ref.at[pl.ds(0, NUM_ROWS), :], su)
    ci = pltpu.make_async_copy(i_ref, io_ref.at[pl.ds(0, NUM_ROWS), :], si)
    cu.start()
    ci.start()
    cu.wait()
    ci.wait()


def _tc_repack(utab, itab):
    return pl.pallas_call(
        _tc_repack_kernel,
        in_specs=[
            pl.BlockSpec(memory_space=pl.ANY),
            pl.BlockSpec(memory_space=pl.ANY),
        ],
        out_specs=[
            pl.BlockSpec(memory_space=pl.ANY),
            pl.BlockSpec(memory_space=pl.ANY),
        ],
        out_shape=[
            jax.ShapeDtypeStruct((PAD_ROWS, 1), jnp.float32),
            jax.ShapeDtypeStruct((PAD_ROWS, 1), jnp.float32),
        ],
        scratch_shapes=[pltpu.SemaphoreType.DMA, pltpu.SemaphoreType.DMA],
    )(utab, itab)


def _tc_dot_kernel(u_ref, i_ref, o_ref):
    o_ref[...] = jnp.sum(u_ref[...] * i_ref[...], axis=0)


def _tc_dot(ufea_t, ifea_t):
    return pl.pallas_call(
        _tc_dot_kernel,
        out_shape=jax.ShapeDtypeStruct((BSZ,), jnp.float32),
    )(ufea_t, ifea_t)


def _tc_combine_kernel(go_ref, dot_ref, ub_ref, ib_ref, o_ref):
    o_ref[...] = dot_ref[...] + ub_ref[...] + ib_ref[...] + go_ref[0]


def _tc_combine(dot, ubias, ibias, globalOffset):
    return pl.pallas_call(
        _tc_combine_kernel,
        out_shape=jax.ShapeDtypeStruct((BSZ,), jnp.float32),
    )(globalOffset, dot, ubias, ibias)


def kernel(batch_userFea, batch_itemFea, batch_uid, batch_iid,
           globalOffset, uid_userOffset, iid_itemOffset):
    utab, itab = lax.optimization_barrier(
        (uid_userOffset.T, iid_itemOffset.T))
    ubias, ibias = _sc_gather(batch_uid.astype(jnp.int32),
                              batch_iid.astype(jnp.int32), utab, itab)
    dot = _tc_dot(batch_userFea.T, batch_itemFea.T)
    out = _tc_combine(dot, ubias, ibias, globalOffset)
    return out.reshape(BSZ, 1)


# speedup vs baseline: 6.0484x; 1.0276x over previous
import jax
import jax.numpy as jnp
from jax import lax
from jax.experimental import pallas as pl
from jax.experimental.pallas import tpu as pltpu
from jax.experimental.pallas import tpu_sc as plsc

BSZ = 16384
FEA = 64
NC = 2
NS = 16
NW = NC * NS
B_PER_W = BSZ // NW
LANES = 16


def _sc_gather_kernel(uid_hbm, iid_hbm, utab_hbm, itab_hbm, out_hbm,
                      uidx_v, iidx_v, urows_v, irows_v,
                      sem_u, sem_i, sem_x, sem_y):
    wid = lax.axis_index("s") * NC + lax.axis_index("c")
    base = wid * B_PER_W
    cp_x = pltpu.async_copy(uid_hbm.at[pl.ds(base, B_PER_W)], uidx_v, sem_x)
    cp_y = pltpu.async_copy(iid_hbm.at[pl.ds(base, B_PER_W)], iidx_v, sem_y)
    cp_x.wait()
    cp_u = pltpu.async_copy(utab_hbm.at[0].at[uidx_v], urows_v, sem_u)
    cp_y.wait()
    cp_i = pltpu.async_copy(itab_hbm.at[0].at[iidx_v], irows_v, sem_i)
    cp_u.wait()
    cp_i.wait()
    for j in range(B_PER_W // LANES):
        sl = pl.ds(j * LANES, LANES)
        urows_v[sl] = urows_v[sl] + irows_v[sl]
    pltpu.sync_copy(urows_v, out_hbm.at[pl.ds(base, B_PER_W)])


def _sc_gather(uid, iid, utab, itab):
    mesh = plsc.VectorSubcoreMesh(
        core_axis_name="c", subcore_axis_name="s",
        num_cores=NC, num_subcores=NS)
    return pl.kernel(
        _sc_gather_kernel,
        out_type=jax.ShapeDtypeStruct((BSZ,), jnp.float32),
        mesh=mesh,
        compiler_params=pltpu.CompilerParams(
            needs_layout_passes=False, use_tc_tiling_on_sc=True),
        scratch_types=[
            pltpu.VMEM((B_PER_W,), jnp.int32),
            pltpu.VMEM((B_PER_W,), jnp.int32),
            pltpu.VMEM((B_PER_W,), jnp.float32),
            pltpu.VMEM((B_PER_W,), jnp.float32),
            pltpu.SemaphoreType.DMA,
            pltpu.SemaphoreType.DMA,
            pltpu.SemaphoreType.DMA,
            pltpu.SemaphoreType.DMA,
        ],
    )(uid, iid, utab, itab)


NUM_ROWS = 1000000
PAD_ROWS = 1000448


def _tc_repack_kernel(u_ref, i_ref, uo_ref, io_ref, su, si):
    cu = pltpu.make_async_copy(u_ref, uo_ref.at[pl.ds(0, NUM_ROWS), :], su)
    ci = pltpu.make_async_copy(i_ref, io_ref.at[pl.ds(0, NUM_ROWS), :], si)
    cu.start()
    ci.start()
    cu.wait()
    ci.wait()


def _tc_repack(utab, itab):
    return pl.pallas_call(
        _tc_repack_kernel,
        in_specs=[
            pl.BlockSpec(memory_space=pl.ANY),
            pl.BlockSpec(memory_space=pl.ANY),
        ],
        out_specs=[
            pl.BlockSpec(memory_space=pl.ANY),
            pl.BlockSpec(memory_space=pl.ANY),
        ],
        out_shape=[
            jax.ShapeDtypeStruct((PAD_ROWS, 1), jnp.float32),
            jax.ShapeDtypeStruct((PAD_ROWS, 1), jnp.float32),
        ],
        scratch_shapes=[pltpu.SemaphoreType.DMA, pltpu.SemaphoreType.DMA],
    )(utab, itab)


def _tc_dot_kernel(u_ref, i_ref, o_ref):
    o_ref[...] = jnp.sum(u_ref[...] * i_ref[...], axis=0)


def _tc_dot(ufea_t, ifea_t):
    return pl.pallas_call(
        _tc_dot_kernel,
        out_shape=jax.ShapeDtypeStruct((BSZ,), jnp.float32),
    )(ufea_t, ifea_t)


def _tc_combine_kernel(go_ref, dot_ref, b_ref, o_ref):
    o_ref[...] = dot_ref[...] + b_ref[...] + go_ref[0]


def _tc_combine(dot, bias, globalOffset):
    return pl.pallas_call(
        _tc_combine_kernel,
        out_shape=jax.ShapeDtypeStruct((BSZ,), jnp.float32),
    )(globalOffset, dot, bias)


def kernel(batch_userFea, batch_itemFea, batch_uid, batch_iid,
           globalOffset, uid_userOffset, iid_itemOffset):
    utab, itab = lax.optimization_barrier(
        (uid_userOffset.T, iid_itemOffset.T))
    bias = _sc_gather(batch_uid.astype(jnp.int32),
                      batch_iid.astype(jnp.int32), utab, itab)
    dot = _tc_dot(batch_userFea.T, batch_itemFea.T)
    out = _tc_combine(dot, bias, globalOffset)
    return out.reshape(BSZ, 1)
